# baseline (speedup 1.0000x reference)
import functools

import jax
import jax.numpy as jnp
from jax import lax
from jax.experimental import pallas as pl
from jax.experimental.pallas import tpu as pltpu
from jax.experimental.pallas import tpu_sc as plsc

XS = YS = ZS = 200
VOL = XS * YS * ZS
NC, NS, L = 2, 16, 16
NPASS = 5
S = VOL // (NC * NPASS)
DUMP = 512
ACCN = S + DUMP
CH = 1024
FB = 4096
IC = 2000


def _f16round(x):
  m = plsc.bitcast(x, jnp.int32)
  bias = jnp.int32(0xFFF) + (jnp.right_shift(m, 13) & jnp.int32(1))
  r = (m + bias) & jnp.int32(-8192)
  y = plsc.bitcast(r, jnp.float32)
  inf = jnp.where(x > 0, jnp.float32(jnp.inf), jnp.float32(-jnp.inf))
  return jnp.where(jnp.abs(x) >= jnp.float32(65520.0), inf, y)


def _lin_body(ix, iy, iz, o):
  o[...] = (ix[...] * YS + iy[...]) * ZS + iz[...]


def _sc_body(lin_hbm, w_hbm, vals_hbm, wvol_hbm, vvol_hbm,
             outw, outv,
             accw, accu, ibw, ibv, ibu, aw, au, linb, wb, vb,
             cidx, cw, cu,
             *, m_pts):
  c = lax.axis_index("c")
  s = lax.axis_index("s")
  pts_per_sub = m_pts // NS
  nchunk = pts_per_sub // CH
  pt0 = s * pts_per_sub
  lanes = lax.iota(jnp.int32, L)
  nci = (S // IC) // NS + jnp.where(s < (S // IC) % NS, 1, 0)

  def clear_cidx(_=None):
    def clr(k, carry):
      cidx[pl.ds(k * L, L)] = jnp.int32(S) + ((k * L + lanes) & (DUMP - 1))
      return carry
    lax.fori_loop(0, FB // L, clr, None)

  clear_cidx()

  def pass_body(p, _):
    pbase = (c * NPASS + p) * S

    def init_chunk(ci, carry):
      off = pl.multiple_of(ci * IC, 8)
      g = pl.multiple_of(pbase + off, 8)
      pltpu.sync_copy(wvol_hbm.at[pl.ds(g, IC)], ibw)
      pltpu.sync_copy(vvol_hbm.at[pl.ds(g, IC)], ibv)
      def mulk(k, cc):
        ibu[pl.ds(k * L, L)] = ibw[pl.ds(k * L, L)] * ibv[pl.ds(k * L, L)]
        return cc
      lax.fori_loop(0, IC // L, mulk, None)
      pltpu.sync_copy(ibw, accw.at[pl.ds(off, IC)])
      pltpu.sync_copy(ibu, accu.at[pl.ds(off, IC)])
      return carry
    lax.fori_loop(0, nci, lambda i, cc: init_chunk(s + i * NS, cc), None)
    plsc.subcore_barrier()

    def flush():
      pltpu.sync_copy(cw, accw.at[cidx], add=True)
      pltpu.sync_copy(cu, accu.at[cidx], add=True)
      clear_cidx()

    def chunk_body(j, off):
      need = off > FB - CH
      @pl.when(need)
      def _():
        flush()
      off = jnp.where(need, 0, off)
      base = pl.multiple_of(pt0 + j * CH, CH)
      pltpu.sync_copy(lin_hbm.at[pl.ds(base, CH)], linb)
      pltpu.sync_copy(w_hbm.at[pl.ds(base, CH)], wb)
      pltpu.sync_copy(vals_hbm.at[pl.ds(pl.multiple_of(base // 8, CH // 8), CH // 8)], vb)
      def grp(k, off2):
        l16 = linb[pl.ds(k * L, L)]
        loc = l16 - pbase
        msk = (loc >= 0) & (loc < S)
        w16 = wb[pl.ds(k * L, L)]
        v16 = plsc.load_gather(vb, [2 * k + jnp.right_shift(lanes, 3)])
        u16 = w16 * v16
        cnt = jnp.sum(msk.astype(jnp.int32))
        plsc.store_compressed(cidx.at[pl.ds(off2, L)], loc, mask=msk)
        plsc.store_compressed(cw.at[pl.ds(off2, L)], w16, mask=msk)
        plsc.store_compressed(cu.at[pl.ds(off2, L)], u16, mask=msk)
        return off2 + cnt
      return lax.fori_loop(0, CH // L, grp, off)
    lax.fori_loop(0, nchunk, chunk_body, jnp.int32(0))
    flush()
    plsc.subcore_barrier()

    def wb_chunk(ci, carry):
      off = pl.multiple_of(ci * IC, 8)
      g = pl.multiple_of(pbase + off, 8)
      pltpu.sync_copy(wvol_hbm.at[pl.ds(g, IC)], ibw)
      pltpu.sync_copy(vvol_hbm.at[pl.ds(g, IC)], ibv)
      pltpu.sync_copy(accw.at[pl.ds(off, IC)], aw)
      pltpu.sync_copy(accu.at[pl.ds(off, IC)], au)
      def blend(k, cc):
        w0 = ibw[pl.ds(k * L, L)]
        v0 = ibv[pl.ds(k * L, L)]
        sw = aw[pl.ds(k * L, L)]
        su = au[pl.ds(k * L, L)]
        touched = sw != w0
        ibw[pl.ds(k * L, L)] = jnp.where(touched, _f16round(sw), w0)
        ibv[pl.ds(k * L, L)] = jnp.where(touched, _f16round(su / sw), v0)
        return cc
      lax.fori_loop(0, IC // L, blend, None)
      pltpu.sync_copy(ibw, outw.at[pl.ds(g, IC)])
      pltpu.sync_copy(ibv, outv.at[pl.ds(g, IC)])
      return carry
    lax.fori_loop(0, nci, lambda i, cc: wb_chunk(s + i * NS, cc), None)
    plsc.subcore_barrier()
    return _

  lax.fori_loop(0, NPASS, pass_body, None)


def kernel(values, indices, weights, values_volume, weights_volume,
           scores_volume, semantics_volume):
  n = values.size
  m = n * 8
  idxr = indices.reshape(m, 3)
  rows = m // 1024
  ix = idxr[:, 0].reshape(rows, 1024)
  iy = idxr[:, 1].reshape(rows, 1024)
  iz = idxr[:, 2].reshape(rows, 1024)
  lin = pl.pallas_call(
      _lin_body,
      out_shape=jax.ShapeDtypeStruct((rows, 1024), jnp.int32),
      grid=(rows // 8,),
      in_specs=[pl.BlockSpec((8, 1024), lambda i: (i, 0))] * 3,
      out_specs=pl.BlockSpec((8, 1024), lambda i: (i, 0)),
  )(ix, iy, iz)

  mesh = plsc.VectorSubcoreMesh(core_axis_name="c", subcore_axis_name="s")
  sc = functools.partial(
      pl.kernel,
      out_type=(
          jax.ShapeDtypeStruct((VOL,), jnp.float32),
          jax.ShapeDtypeStruct((VOL,), jnp.float32),
      ),
      mesh=mesh,
      compiler_params=pltpu.CompilerParams(needs_layout_passes=False),
      scratch_types=[
          pltpu.VMEM_SHARED((ACCN,), jnp.float32),
          pltpu.VMEM_SHARED((ACCN,), jnp.float32),
          pltpu.VMEM((IC,), jnp.float32),
          pltpu.VMEM((IC,), jnp.float32),
          pltpu.VMEM((IC,), jnp.float32),
          pltpu.VMEM((IC,), jnp.float32),
          pltpu.VMEM((IC,), jnp.float32),
          pltpu.VMEM((CH,), jnp.int32),
          pltpu.VMEM((CH,), jnp.float32),
          pltpu.VMEM((CH // 8,), jnp.float32),
          pltpu.VMEM((FB,), jnp.int32),
          pltpu.VMEM((FB,), jnp.float32),
          pltpu.VMEM((FB,), jnp.float32),
      ],
  )(functools.partial(_sc_body, m_pts=m))

  outw, outv = sc(
      lin.reshape(m),
      weights.reshape(m).astype(jnp.float32),
      values.reshape(n).astype(jnp.float32),
      weights_volume.reshape(VOL),
      values_volume.reshape(VOL),
  )
  return (outv.reshape(XS, YS, ZS), outw.reshape(XS, YS, ZS),
          semantics_volume, scores_volume)


# speedup vs baseline: 1.4704x; 1.4704x over previous
import functools

import jax
import jax.numpy as jnp
from jax import lax
from jax.experimental import pallas as pl
from jax.experimental.pallas import tpu as pltpu
from jax.experimental.pallas import tpu_sc as plsc

XS = YS = ZS = 200
VOL = XS * YS * ZS
NC, NS, L = 2, 16, 16
NPASS = 5
S = VOL // (NC * NPASS)
DUMP = 512
ACCN = S + DUMP
CH = 1024
FB = 4096
IC = 1000


def _f16round(x):
  m = plsc.bitcast(x, jnp.int32)
  bias = jnp.int32(0xFFF) + (jnp.right_shift(m, 13) & jnp.int32(1))
  r = (m + bias) & jnp.int32(-8192)
  y = plsc.bitcast(r, jnp.float32)
  inf = jnp.where(x > 0, jnp.float32(jnp.inf), jnp.float32(-jnp.inf))
  return jnp.where(jnp.abs(x) >= jnp.float32(65520.0), inf, y)


def _lin_body(ix, iy, iz, o):
  o[...] = (ix[...] * YS + iy[...]) * ZS + iz[...]


def _sc_body(lin_hbm, w_hbm, vals_hbm, wvol_hbm, vvol_hbm,
             outw, outv,
             accw, accu,
             ibw0, ibw1, ibv0, ibv1, obw0, obw1, obv0, obv1, aw, au,
             linb0, linb1, wb0, wb1, vb0, vb1,
             cidx, cw, cu,
             sio0, sio1, soo0, soo1, ssc0, ssc1,
             *, m_pts):
  c = lax.axis_index("c")
  s = lax.axis_index("s")
  pts_per_sub = m_pts // NS
  nchunk = pts_per_sub // CH
  pt0 = s * pts_per_sub
  lanes = lax.iota(jnp.int32, L)
  ibw = (ibw0, ibw1)
  ibv = (ibv0, ibv1)
  obw = (obw0, obw1)
  obv = (obv0, obv1)
  linb = (linb0, linb1)
  wbuf = (wb0, wb1)
  vbuf = (vb0, vb1)
  sio = (sio0, sio1)
  soo = (soo0, soo1)
  ssc = (ssc0, ssc1)
  nci = (S // IC) // NS

  def clear_cidx(_=None):
    def clr(k, carry):
      cidx[pl.ds(k * L, L)] = jnp.int32(S) + ((k * L + lanes) & (DUMP - 1))
      return carry
    lax.fori_loop(0, FB // L, clr, None)

  clear_cidx()

  def pass_body(p, _):
    pbase = (c * NPASS + p) * S

    def slice_g(i):
      ci = s + i * NS
      off = pl.multiple_of(ci * IC, 8)
      return off, pl.multiple_of(pbase + off, 8)

    def issue_slice_in(b, i):
      off, g = slice_g(i)
      pltpu.async_copy(wvol_hbm.at[pl.ds(g, IC)], ibw[b], sio[b])
      pltpu.async_copy(vvol_hbm.at[pl.ds(g, IC)], ibv[b], sio[b])

    def wait_slice_in(b):
      pltpu.make_async_copy(wvol_hbm.at[pl.ds(0, IC)], ibw[b], sio[b]).wait()
      pltpu.make_async_copy(vvol_hbm.at[pl.ds(0, IC)], ibv[b], sio[b]).wait()

    issue_slice_in(0, 0)
    issue_slice_in(1, 1)
    def init_pair(ii, carry):
      for b in (0, 1):
        i = ii * 2 + b
        wait_slice_in(b)
        def mulk(k, cc):
          obw[b][pl.ds(k * L, L)] = (
              ibw[b][pl.ds(k * L, L)] * ibv[b][pl.ds(k * L, L)])
          return cc
        lax.fori_loop(0, IC // L, mulk, None)
        off, g = slice_g(i)
        pltpu.sync_copy(ibw[b], accw.at[pl.ds(off, IC)])
        pltpu.sync_copy(obw[b], accu.at[pl.ds(off, IC)])
        @pl.when(i + 2 < nci)
        def _():
          issue_slice_in(b, i + 2)
      return carry
    lax.fori_loop(0, nci // 2, init_pair, None)
    plsc.subcore_barrier()

    def flush():
      pltpu.sync_copy(cw, accw.at[cidx], add=True)
      pltpu.sync_copy(cu, accu.at[cidx], add=True)
      clear_cidx()

    def issue_scan(b, j):
      base = pl.multiple_of(pt0 + j * CH, CH)
      pltpu.async_copy(lin_hbm.at[pl.ds(base, CH)], linb[b], ssc[b])
      pltpu.async_copy(w_hbm.at[pl.ds(base, CH)], wbuf[b], ssc[b])
      pltpu.async_copy(
          vals_hbm.at[pl.ds(pl.multiple_of(base // 8, CH // 8), CH // 8)],
          vbuf[b], ssc[b])

    def wait_scan(b):
      pltpu.make_async_copy(lin_hbm.at[pl.ds(0, CH)], linb[b], ssc[b]).wait()
      pltpu.make_async_copy(w_hbm.at[pl.ds(0, CH)], wbuf[b], ssc[b]).wait()
      pltpu.make_async_copy(
          vals_hbm.at[pl.ds(0, CH // 8)], vbuf[b], ssc[b]).wait()

    issue_scan(0, 0)
    issue_scan(1, 1)
    def chunk_pair(jj, off):
      for b in (0, 1):
        j = jj * 2 + b
        need = off > FB - CH
        @pl.when(need)
        def _():
          flush()
        off = jnp.where(need, 0, off)
        wait_scan(b)
        def grp(k, off2):
          l16 = linb[b][pl.ds(k * L, L)]
          loc = l16 - pbase
          msk = (loc >= 0) & (loc < S)
          w16 = wbuf[b][pl.ds(k * L, L)]
          v16 = plsc.load_gather(vbuf[b], [2 * k + jnp.right_shift(lanes, 3)])
          u16 = w16 * v16
          cnt = jnp.sum(msk.astype(jnp.int32))
          plsc.store_compressed(cidx.at[pl.ds(off2, L)], loc, mask=msk)
          plsc.store_compressed(cw.at[pl.ds(off2, L)], w16, mask=msk)
          plsc.store_compressed(cu.at[pl.ds(off2, L)], u16, mask=msk)
          return off2 + cnt
        off = lax.fori_loop(0, CH // L, grp, off)
        @pl.when(j + 2 < nchunk)
        def _():
          issue_scan(b, j + 2)
      return off
    lax.fori_loop(0, nchunk // 2, chunk_pair, jnp.int32(0))
    flush()
    plsc.subcore_barrier()

    issue_slice_in(0, 0)
    issue_slice_in(1, 1)
    def wb_pair(ii, carry):
      for b in (0, 1):
        i = ii * 2 + b
        off, g = slice_g(i)
        wait_slice_in(b)
        @pl.when(i >= 2)
        def _():
          pltpu.make_async_copy(obw[b], outw.at[pl.ds(g, IC)], soo[b]).wait()
          pltpu.make_async_copy(obv[b], outv.at[pl.ds(g, IC)], soo[b]).wait()
        pltpu.sync_copy(accw.at[pl.ds(off, IC)], aw)
        pltpu.sync_copy(accu.at[pl.ds(off, IC)], au)
        def blend(k, cc):
          w0 = ibw[b][pl.ds(k * L, L)]
          v0 = ibv[b][pl.ds(k * L, L)]
          sw = aw[pl.ds(k * L, L)]
          su = au[pl.ds(k * L, L)]
          touched = sw != w0
          obw[b][pl.ds(k * L, L)] = jnp.where(touched, _f16round(sw), w0)
          obv[b][pl.ds(k * L, L)] = jnp.where(touched, _f16round(su / sw), v0)
          return cc
        lax.fori_loop(0, IC // L, blend, None)
        pltpu.async_copy(obw[b], outw.at[pl.ds(g, IC)], soo[b])
        pltpu.async_copy(obv[b], outv.at[pl.ds(g, IC)], soo[b])
        @pl.when(i + 2 < nci)
        def _():
          issue_slice_in(b, i + 2)
      return carry
    lax.fori_loop(0, nci // 2, wb_pair, None)
    for b in (0, 1):
      pltpu.make_async_copy(obw[b], outw.at[pl.ds(0, IC)], soo[b]).wait()
      pltpu.make_async_copy(obv[b], outv.at[pl.ds(0, IC)], soo[b]).wait()
    plsc.subcore_barrier()
    return _

  lax.fori_loop(0, NPASS, pass_body, None)


def kernel(values, indices, weights, values_volume, weights_volume,
           scores_volume, semantics_volume):
  n = values.size
  m = n * 8
  idxr = indices.reshape(m, 3)
  rows = m // 1024
  ix = idxr[:, 0].reshape(rows, 1024)
  iy = idxr[:, 1].reshape(rows, 1024)
  iz = idxr[:, 2].reshape(rows, 1024)
  lin = pl.pallas_call(
      _lin_body,
      out_shape=jax.ShapeDtypeStruct((rows, 1024), jnp.int32),
      grid=(rows // 8,),
      in_specs=[pl.BlockSpec((8, 1024), lambda i: (i, 0))] * 3,
      out_specs=pl.BlockSpec((8, 1024), lambda i: (i, 0)),
  )(ix, iy, iz)

  mesh = plsc.VectorSubcoreMesh(core_axis_name="c", subcore_axis_name="s")
  sc = functools.partial(
      pl.kernel,
      out_type=(
          jax.ShapeDtypeStruct((VOL,), jnp.float32),
          jax.ShapeDtypeStruct((VOL,), jnp.float32),
      ),
      mesh=mesh,
      compiler_params=pltpu.CompilerParams(needs_layout_passes=False),
      scratch_types=[
          pltpu.VMEM_SHARED((ACCN,), jnp.float32),
          pltpu.VMEM_SHARED((ACCN,), jnp.float32),
          pltpu.VMEM((IC,), jnp.float32),
          pltpu.VMEM((IC,), jnp.float32),
          pltpu.VMEM((IC,), jnp.float32),
          pltpu.VMEM((IC,), jnp.float32),
          pltpu.VMEM((IC,), jnp.float32),
          pltpu.VMEM((IC,), jnp.float32),
          pltpu.VMEM((IC,), jnp.float32),
          pltpu.VMEM((IC,), jnp.float32),
          pltpu.VMEM((IC,), jnp.float32),
          pltpu.VMEM((IC,), jnp.float32),
          pltpu.VMEM((CH,), jnp.int32),
          pltpu.VMEM((CH,), jnp.int32),
          pltpu.VMEM((CH,), jnp.float32),
          pltpu.VMEM((CH,), jnp.float32),
          pltpu.VMEM((CH // 8,), jnp.float32),
          pltpu.VMEM((CH // 8,), jnp.float32),
          pltpu.VMEM((FB,), jnp.int32),
          pltpu.VMEM((FB,), jnp.float32),
          pltpu.VMEM((FB,), jnp.float32),
          pltpu.SemaphoreType.DMA,
          pltpu.SemaphoreType.DMA,
          pltpu.SemaphoreType.DMA,
          pltpu.SemaphoreType.DMA,
          pltpu.SemaphoreType.DMA,
          pltpu.SemaphoreType.DMA,
      ],
  )(functools.partial(_sc_body, m_pts=m))

  outw, outv = sc(
      lin.reshape(m),
      weights.reshape(m).astype(jnp.float32),
      values.reshape(n).astype(jnp.float32),
      weights_volume.reshape(VOL),
      values_volume.reshape(VOL),
  )
  return (outv.reshape(XS, YS, ZS), outw.reshape(XS, YS, ZS),
          semantics_volume, scores_volume)


# speedup vs baseline: 1.4946x; 1.0165x over previous
import functools

import jax
import jax.numpy as jnp
from jax import lax
from jax.experimental import pallas as pl
from jax.experimental.pallas import tpu as pltpu
from jax.experimental.pallas import tpu_sc as plsc

XS = YS = ZS = 200
VOL = XS * YS * ZS
NC, NS, L = 2, 16, 16
NPASS = 5
S = VOL // (NC * NPASS)
DUMP = 512
ACCN = S + DUMP
CH = 1024
FB = 4096
IC = 2000


def _f16round(x):
  m = plsc.bitcast(x, jnp.int32)
  bias = jnp.int32(0xFFF) + (jnp.right_shift(m, 13) & jnp.int32(1))
  r = (m + bias) & jnp.int32(-8192)
  y = plsc.bitcast(r, jnp.float32)
  inf = jnp.where(x > 0, jnp.float32(jnp.inf), jnp.float32(-jnp.inf))
  return jnp.where(jnp.abs(x) >= jnp.float32(65520.0), inf, y)


def _lin_body(ix, iy, iz, o):
  o[...] = (ix[...] * YS + iy[...]) * ZS + iz[...]


def _sc_body(lin_hbm, w_hbm, vals_hbm, wvol_hbm, vvol_hbm,
             outw, outv,
             accw, accu,
             ibw0, ibw1, ibv0, ibv1, aw, au,
             linb0, linb1, wb0, wb1, vb0, vb1,
             cidx, cw, cu,
             sio0, sio1, ssc0, ssc1,
             *, m_pts):
  c = lax.axis_index("c")
  s = lax.axis_index("s")
  pts_per_sub = m_pts // NS
  nchunk = pts_per_sub // CH
  pt0 = s * pts_per_sub
  lanes = lax.iota(jnp.int32, L)
  ibw = (ibw0, ibw1)
  ibv = (ibv0, ibv1)
  linb = (linb0, linb1)
  wbuf = (wb0, wb1)
  vbuf = (vb0, vb1)
  sio = (sio0, sio1)
  ssc = (ssc0, ssc1)
  nci = (S // IC) // NS

  def clear_cidx(_=None):
    def clr(k, carry):
      cidx[pl.ds(k * L, L)] = jnp.int32(S) + ((k * L + lanes) & (DUMP - 1))
      return carry
    lax.fori_loop(0, FB // L, clr, None)

  clear_cidx()

  def pass_body(p, _):
    pbase = (c * NPASS + p) * S

    def slice_g(i):
      ci = s + i * NS
      off = pl.multiple_of(ci * IC, 8)
      return off, pl.multiple_of(pbase + off, 8)

    def issue_slice_in(b, i):
      off, g = slice_g(i)
      pltpu.async_copy(wvol_hbm.at[pl.ds(g, IC)], ibw[b], sio[b])
      pltpu.async_copy(vvol_hbm.at[pl.ds(g, IC)], ibv[b], sio[b])

    def wait_slice_in(b):
      pltpu.make_async_copy(wvol_hbm.at[pl.ds(0, IC)], ibw[b], sio[b]).wait()
      pltpu.make_async_copy(vvol_hbm.at[pl.ds(0, IC)], ibv[b], sio[b]).wait()

    def init_body(b, i):
      wait_slice_in(b)
      def mulk(k, cc):
        aw[pl.ds(k * L, L)] = (
            ibw[b][pl.ds(k * L, L)] * ibv[b][pl.ds(k * L, L)])
        return cc
      lax.fori_loop(0, IC // L, mulk, None)
      off, g = slice_g(i)
      pltpu.sync_copy(ibw[b], accw.at[pl.ds(off, IC)])
      pltpu.sync_copy(aw, accu.at[pl.ds(off, IC)])
      @pl.when(i + 2 < nci)
      def _():
        issue_slice_in(b, i + 2)

    issue_slice_in(0, 0)
    issue_slice_in(1, 1)
    def init_pair(ii, carry):
      for b in (0, 1):
        init_body(b, ii * 2 + b)
      return carry
    lax.fori_loop(0, nci // 2, init_pair, None)
    init_body(0, nci - 1)
    plsc.subcore_barrier()

    def flush():
      pltpu.sync_copy(cw, accw.at[cidx], add=True)
      pltpu.sync_copy(cu, accu.at[cidx], add=True)
      clear_cidx()

    def issue_scan(b, j):
      base = pl.multiple_of(pt0 + j * CH, CH)
      pltpu.async_copy(lin_hbm.at[pl.ds(base, CH)], linb[b], ssc[b])
      pltpu.async_copy(w_hbm.at[pl.ds(base, CH)], wbuf[b], ssc[b])
      pltpu.async_copy(
          vals_hbm.at[pl.ds(pl.multiple_of(base // 8, CH // 8), CH // 8)],
          vbuf[b], ssc[b])

    def wait_scan(b):
      pltpu.make_async_copy(lin_hbm.at[pl.ds(0, CH)], linb[b], ssc[b]).wait()
      pltpu.make_async_copy(w_hbm.at[pl.ds(0, CH)], wbuf[b], ssc[b]).wait()
      pltpu.make_async_copy(
          vals_hbm.at[pl.ds(0, CH // 8)], vbuf[b], ssc[b]).wait()

    issue_scan(0, 0)
    issue_scan(1, 1)
    def chunk_pair(jj, off):
      for b in (0, 1):
        j = jj * 2 + b
        need = off > FB - CH
        @pl.when(need)
        def _():
          flush()
        off = jnp.where(need, 0, off)
        wait_scan(b)
        def grp(k, off2):
          l16 = linb[b][pl.ds(k * L, L)]
          loc = l16 - pbase
          msk = (loc >= 0) & (loc < S)
          w16 = wbuf[b][pl.ds(k * L, L)]
          v16 = plsc.load_gather(vbuf[b], [2 * k + jnp.right_shift(lanes, 3)])
          u16 = w16 * v16
          cnt = jnp.sum(msk.astype(jnp.int32))
          plsc.store_compressed(cidx.at[pl.ds(off2, L)], loc, mask=msk)
          plsc.store_compressed(cw.at[pl.ds(off2, L)], w16, mask=msk)
          plsc.store_compressed(cu.at[pl.ds(off2, L)], u16, mask=msk)
          return off2 + cnt
        off = lax.fori_loop(0, CH // L, grp, off)
        @pl.when(j + 2 < nchunk)
        def _():
          issue_scan(b, j + 2)
      return off
    lax.fori_loop(0, nchunk // 2, chunk_pair, jnp.int32(0))
    flush()
    plsc.subcore_barrier()

    def wb_body(b, i):
      off, g = slice_g(i)
      wait_slice_in(b)
      pltpu.sync_copy(accw.at[pl.ds(off, IC)], aw)
      pltpu.sync_copy(accu.at[pl.ds(off, IC)], au)
      def blend(k, cc):
        w0 = ibw[b][pl.ds(k * L, L)]
        v0 = ibv[b][pl.ds(k * L, L)]
        sw = aw[pl.ds(k * L, L)]
        su = au[pl.ds(k * L, L)]
        touched = sw != w0
        ibw[b][pl.ds(k * L, L)] = jnp.where(touched, _f16round(sw), w0)
        ibv[b][pl.ds(k * L, L)] = jnp.where(touched, _f16round(su / sw), v0)
        return cc
      lax.fori_loop(0, IC // L, blend, None)
      pltpu.sync_copy(ibw[b], outw.at[pl.ds(g, IC)])
      pltpu.sync_copy(ibv[b], outv.at[pl.ds(g, IC)])
      @pl.when(i + 2 < nci)
      def _():
        issue_slice_in(b, i + 2)

    issue_slice_in(0, 0)
    issue_slice_in(1, 1)
    def wb_pair(ii, carry):
      for b in (0, 1):
        wb_body(b, ii * 2 + b)
      return carry
    lax.fori_loop(0, nci // 2, wb_pair, None)
    wb_body(0, nci - 1)
    plsc.subcore_barrier()
    return _

  lax.fori_loop(0, NPASS, pass_body, None)


def kernel(values, indices, weights, values_volume, weights_volume,
           scores_volume, semantics_volume):
  n = values.size
  m = n * 8
  idxr = indices.reshape(m, 3)
  rows = m // 1024
  ix = idxr[:, 0].reshape(rows, 1024)
  iy = idxr[:, 1].reshape(rows, 1024)
  iz = idxr[:, 2].reshape(rows, 1024)
  lin = pl.pallas_call(
      _lin_body,
      out_shape=jax.ShapeDtypeStruct((rows, 1024), jnp.int32),
      grid=(rows // 8,),
      in_specs=[pl.BlockSpec((8, 1024), lambda i: (i, 0))] * 3,
      out_specs=pl.BlockSpec((8, 1024), lambda i: (i, 0)),
  )(ix, iy, iz)

  mesh = plsc.VectorSubcoreMesh(core_axis_name="c", subcore_axis_name="s")
  sc = functools.partial(
      pl.kernel,
      out_type=(
          jax.ShapeDtypeStruct((VOL,), jnp.float32),
          jax.ShapeDtypeStruct((VOL,), jnp.float32),
      ),
      mesh=mesh,
      compiler_params=pltpu.CompilerParams(needs_layout_passes=False),
      scratch_types=[
          pltpu.VMEM_SHARED((ACCN,), jnp.float32),
          pltpu.VMEM_SHARED((ACCN,), jnp.float32),
          pltpu.VMEM((IC,), jnp.float32),
          pltpu.VMEM((IC,), jnp.float32),
          pltpu.VMEM((IC,), jnp.float32),
          pltpu.VMEM((IC,), jnp.float32),
          pltpu.VMEM((IC,), jnp.float32),
          pltpu.VMEM((IC,), jnp.float32),
          pltpu.VMEM((CH,), jnp.int32),
          pltpu.VMEM((CH,), jnp.int32),
          pltpu.VMEM((CH,), jnp.float32),
          pltpu.VMEM((CH,), jnp.float32),
          pltpu.VMEM((CH // 8,), jnp.float32),
          pltpu.VMEM((CH // 8,), jnp.float32),
          pltpu.VMEM((FB,), jnp.int32),
          pltpu.VMEM((FB,), jnp.float32),
          pltpu.VMEM((FB,), jnp.float32),
          pltpu.SemaphoreType.DMA,
          pltpu.SemaphoreType.DMA,
          pltpu.SemaphoreType.DMA,
          pltpu.SemaphoreType.DMA,
      ],
  )(functools.partial(_sc_body, m_pts=m))

  outw, outv = sc(
      lin.reshape(m),
      weights.reshape(m).astype(jnp.float32),
      values.reshape(n).astype(jnp.float32),
      weights_volume.reshape(VOL),
      values_volume.reshape(VOL),
  )
  return (outv.reshape(XS, YS, ZS), outw.reshape(XS, YS, ZS),
          semantics_volume, scores_volume)
